# Initial kernel scaffold; baseline (speedup 1.0000x reference)
import functools

import jax
import jax.numpy as jnp
from jax import lax
from jax.experimental import pallas as pl
from jax.experimental.pallas import tpu as pltpu
from jax.experimental.pallas import tpu_sc as plsc

N_FEAT = 26
BATCH = 16384
ROWS = 5
DIM = 3
OUT_D = N_FEAT * DIM
NC, NS, LANES = 2, 16, 16
NW = NC * NS
B_TILE = BATCH // NW
NVEC = B_TILE // LANES
TBL_WORDS = N_FEAT * ROWS * DIM
TBL_PAD = 400

_mesh = plsc.VectorSubcoreMesh(
    core_axis_name="c", subcore_axis_name="s", num_cores=NC, num_subcores=NS
)


@functools.partial(
    pl.kernel,
    out_type=jax.ShapeDtypeStruct((BATCH * OUT_D,), jnp.float32),
    mesh=_mesh,
    scratch_types=[
        pltpu.VMEM((N_FEAT, B_TILE), jnp.int32),
        pltpu.VMEM((TBL_PAD,), jnp.float32),
        pltpu.VMEM((B_TILE * OUT_D,), jnp.float32),
        pltpu.SemaphoreType.DMA,
    ],
)
def _embed_sc(*refs):
    idx_hbm = refs[:N_FEAT]
    tbl_hbm = refs[N_FEAT]
    out_hbm = refs[N_FEAT + 1]
    idx_v, tbl_v, out_v, sem = refs[N_FEAT + 2:]

    wid = lax.axis_index("s") * NC + lax.axis_index("c")
    base = wid * B_TILE

    copies = [
        pltpu.async_copy(idx_hbm[i].at[pl.ds(base, B_TILE)], idx_v.at[i], sem)
        for i in range(N_FEAT)
    ]
    pltpu.sync_copy(tbl_hbm, tbl_v)
    for c in copies:
        c.wait()

    lane = lax.broadcasted_iota(jnp.int32, (LANES,), 0)
    lane_out = lane * OUT_D

    def body(j, carry):
        ob_base = lane_out + j * (LANES * OUT_D)
        for i in range(N_FEAT):
            idx16 = idx_v[i, pl.ds(j * LANES, LANES)]
            a3 = idx16 * DIM + (i * ROWS * DIM)
            for d in range(DIM):
                val = plsc.load_gather(tbl_v, [a3 + d])
                plsc.store_scatter(out_v, [ob_base + (i * DIM + d)], val)
        return carry

    lax.fori_loop(0, NVEC, body, 0)

    pltpu.sync_copy(out_v, out_hbm.at[pl.ds(base * OUT_D, B_TILE * OUT_D)])


def kernel(f0, f1, f2, f3, f4, f5, f6, f7, f8, f9, f10, f11, f12, f13, f14,
           f15, f16, f17, f18, f19, f20, f21, f22, f23, f24, f25,
           W_f0, W_f1, W_f2, W_f3, W_f4, W_f5, W_f6, W_f7, W_f8, W_f9,
           W_f10, W_f11, W_f12, W_f13, W_f14, W_f15, W_f16, W_f17, W_f18,
           W_f19, W_f20, W_f21, W_f22, W_f23, W_f24, W_f25):
    fs = (f0, f1, f2, f3, f4, f5, f6, f7, f8, f9, f10, f11, f12, f13, f14,
          f15, f16, f17, f18, f19, f20, f21, f22, f23, f24, f25)
    Ws = (W_f0, W_f1, W_f2, W_f3, W_f4, W_f5, W_f6, W_f7, W_f8, W_f9,
          W_f10, W_f11, W_f12, W_f13, W_f14, W_f15, W_f16, W_f17, W_f18,
          W_f19, W_f20, W_f21, W_f22, W_f23, W_f24, W_f25)
    idx = [jnp.asarray(f, jnp.int32) for f in fs]
    tbl = jnp.concatenate(
        [w.reshape(-1).astype(jnp.float32) for w in Ws]
        + [jnp.zeros((TBL_PAD - TBL_WORDS,), jnp.float32)]
    )
    out_flat = _embed_sc(*idx, tbl)
    return out_flat.reshape(BATCH, OUT_D)


# speedup vs baseline: 21.7544x; 21.7544x over previous
import functools

import jax
import jax.numpy as jnp
from jax import lax
from jax.experimental import pallas as pl
from jax.experimental.pallas import tpu as pltpu
from jax.experimental.pallas import tpu_sc as plsc

N_FEAT = 26
BATCH = 16384
ROWS = 5
DIM = 3
OUT_D = N_FEAT * DIM
NC, NS, LANES = 2, 16, 16
NW = NC * NS
B_TILE = BATCH // NW
NVEC = B_TILE // LANES
TBL_WORDS = N_FEAT * ROWS * DIM
TBL_PAD = 400

_mesh = plsc.VectorSubcoreMesh(
    core_axis_name="c", subcore_axis_name="s", num_cores=NC, num_subcores=NS
)


@functools.partial(
    pl.kernel,
    out_type=jax.ShapeDtypeStruct((BATCH * OUT_D,), jnp.float32),
    mesh=_mesh,
    scratch_types=[
        pltpu.VMEM((N_FEAT, B_TILE), jnp.int32),
        pltpu.VMEM((TBL_PAD,), jnp.float32),
        pltpu.VMEM((B_TILE * OUT_D,), jnp.float32),
        pltpu.SemaphoreType.DMA,
    ],
    compiler_params=pltpu.CompilerParams(needs_layout_passes=False),
)
def _embed_sc(*refs):
    idx_hbm = refs[:N_FEAT]
    tbl_hbm = refs[N_FEAT]
    out_hbm = refs[N_FEAT + 1]
    idx_v, tbl_v, out_v, sem = refs[N_FEAT + 2:]

    wid = lax.axis_index("s") * NC + lax.axis_index("c")
    base = wid * B_TILE

    copies = [
        pltpu.async_copy(idx_hbm[i].at[pl.ds(base, B_TILE)], idx_v.at[i], sem)
        for i in range(N_FEAT)
    ]
    pltpu.sync_copy(tbl_hbm, tbl_v)
    for c in copies:
        c.wait()

    lane = lax.broadcasted_iota(jnp.int32, (LANES,), 0)
    lane_out = lane * OUT_D

    def body(j, carry):
        ob_base = lane_out + j * (LANES * OUT_D)
        for i in range(N_FEAT):
            idx16 = idx_v[i, pl.ds(j * LANES, LANES)]
            a3 = idx16 * DIM + (i * ROWS * DIM)
            for d in range(DIM):
                val = plsc.load_gather(tbl_v, [a3 + d])
                plsc.store_scatter(out_v, [ob_base + (i * DIM + d)], val)
        return carry

    lax.fori_loop(0, NVEC, body, 0)

    pltpu.sync_copy(out_v, out_hbm.at[pl.ds(base * OUT_D, B_TILE * OUT_D)])


def kernel(f0, f1, f2, f3, f4, f5, f6, f7, f8, f9, f10, f11, f12, f13, f14,
           f15, f16, f17, f18, f19, f20, f21, f22, f23, f24, f25,
           W_f0, W_f1, W_f2, W_f3, W_f4, W_f5, W_f6, W_f7, W_f8, W_f9,
           W_f10, W_f11, W_f12, W_f13, W_f14, W_f15, W_f16, W_f17, W_f18,
           W_f19, W_f20, W_f21, W_f22, W_f23, W_f24, W_f25):
    fs = (f0, f1, f2, f3, f4, f5, f6, f7, f8, f9, f10, f11, f12, f13, f14,
          f15, f16, f17, f18, f19, f20, f21, f22, f23, f24, f25)
    Ws = (W_f0, W_f1, W_f2, W_f3, W_f4, W_f5, W_f6, W_f7, W_f8, W_f9,
          W_f10, W_f11, W_f12, W_f13, W_f14, W_f15, W_f16, W_f17, W_f18,
          W_f19, W_f20, W_f21, W_f22, W_f23, W_f24, W_f25)
    idx = [jnp.asarray(f, jnp.int32) for f in fs]
    tbl = jnp.concatenate(
        [w.reshape(-1).astype(jnp.float32) for w in Ws]
        + [jnp.zeros((TBL_PAD - TBL_WORDS,), jnp.float32)]
    )
    out_flat = _embed_sc(*idx, tbl)
    return out_flat.reshape(BATCH, OUT_D)
